# baseline (speedup 1.0000x reference)
import jax
import jax.numpy as jnp
from jax.experimental import pallas as pl
from jax.experimental.pallas import tpu as pltpu

BM1, BN1 = 1024, 512
BM2, BN2 = 1024, 1024


def _quantize(b_ref, c_ref, y):
    def bsel(m, hi, lo):
        return jnp.where(m, hi, lo)

    m3 = y > b_ref[7]
    m2 = y > bsel(m3, b_ref[11], b_ref[3])
    m1 = y > bsel(m3, bsel(m2, b_ref[13], b_ref[9]),
                  bsel(m2, b_ref[5], b_ref[1]))
    m0 = y > bsel(m3, bsel(m2, bsel(m1, b_ref[14], b_ref[12]),
                           bsel(m1, b_ref[10], b_ref[8])),
                  bsel(m2, bsel(m1, b_ref[6], b_ref[4]),
                       bsel(m1, b_ref[2], b_ref[0])))
    idx = (jnp.where(m3, 8, 0) | jnp.where(m2, 4, 0)
           | jnp.where(m1, 2, 0) | jnp.where(m0, 1, 0))
    yhat = bsel(m3, bsel(m2, bsel(m1, bsel(m0, c_ref[15], c_ref[14]),
                                  bsel(m0, c_ref[13], c_ref[12])),
                         bsel(m1, bsel(m0, c_ref[11], c_ref[10]),
                              bsel(m0, c_ref[9], c_ref[8]))),
                bsel(m2, bsel(m1, bsel(m0, c_ref[7], c_ref[6]),
                              bsel(m0, c_ref[5], c_ref[4])),
                     bsel(m1, bsel(m0, c_ref[3], c_ref[2]),
                          bsel(m0, c_ref[1], c_ref[0]))))
    return idx, yhat.astype(jnp.bfloat16)


def _quant_matmul_kernel(b_ref, c_ref, x_ref, pi_ref, idx_ref, yhat_ref):
    h = pi_ref.shape[1] // 2
    x = x_ref[...]
    y0 = jnp.dot(x, pi_ref[:, :h], preferred_element_type=jnp.float32)
    y1 = jnp.dot(x, pi_ref[:, h:], preferred_element_type=jnp.float32)
    idx0, yhat0 = _quantize(b_ref, c_ref, y0)
    idx_ref[:, :h] = idx0
    yhat_ref[:, :h] = yhat0
    idx1, yhat1 = _quantize(b_ref, c_ref, y1)
    idx_ref[:, h:] = idx1
    yhat_ref[:, h:] = yhat1


def _unrotate_kernel(yhat_ref, pi_ref, out_ref):
    out_ref[...] = jax.lax.dot_general(
        yhat_ref[...], pi_ref[...],
        dimension_numbers=(((1,), (1,)), ((), ())),
        preferred_element_type=jnp.float32,
    )


def kernel(x, Pi, centroids, boundaries):
    M, d = x.shape

    pi_bf = Pi.astype(jnp.bfloat16)

    idx, yhat = pl.pallas_call(
        _quant_matmul_kernel,
        grid=(M // BM1, d // BN1),
        in_specs=[
            pl.BlockSpec(memory_space=pltpu.SMEM),
            pl.BlockSpec(memory_space=pltpu.SMEM),
            pl.BlockSpec((BM1, d), lambda i, j: (i, 0)),
            pl.BlockSpec((d, BN1), lambda i, j: (0, j)),
        ],
        out_specs=[
            pl.BlockSpec((BM1, BN1), lambda i, j: (i, j)),
            pl.BlockSpec((BM1, BN1), lambda i, j: (i, j)),
        ],
        out_shape=[
            jax.ShapeDtypeStruct((M, d), jnp.int32),
            jax.ShapeDtypeStruct((M, d), jnp.bfloat16),
        ],
        compiler_params=pltpu.CompilerParams(
            dimension_semantics=("parallel", "arbitrary"),
        ),
    )(boundaries, centroids, x, pi_bf)

    x_hat = pl.pallas_call(
        _unrotate_kernel,
        grid=(M // BM2, d // BN2),
        in_specs=[
            pl.BlockSpec((BM2, d), lambda i, j: (i, 0)),
            pl.BlockSpec((BN2, d), lambda i, j: (j, 0)),
        ],
        out_specs=pl.BlockSpec((BM2, BN2), lambda i, j: (i, j)),
        out_shape=jax.ShapeDtypeStruct((M, d), jnp.float32),
        compiler_params=pltpu.CompilerParams(
            dimension_semantics=("parallel", "arbitrary"),
        ),
    )(yhat, pi_bf)

    return (x_hat, idx)


# speedup vs baseline: 1.0025x; 1.0025x over previous
import jax
import jax.numpy as jnp
from jax.experimental import pallas as pl
from jax.experimental.pallas import tpu as pltpu

BM1, BN1 = 1024, 512
BM2, BN2 = 1024, 1024


def _quant_matmul_kernel(b_ref, c_ref, x_ref, pi_ref, idx_ref, yhat_ref):
    y = jnp.dot(x_ref[...], pi_ref[...], preferred_element_type=jnp.float32)

    def bsel(m, hi, lo):
        return jnp.where(m, hi, lo)

    m3 = y > b_ref[7]
    m2 = y > bsel(m3, b_ref[11], b_ref[3])
    m1 = y > bsel(m3, bsel(m2, b_ref[13], b_ref[9]),
                  bsel(m2, b_ref[5], b_ref[1]))
    m0 = y > bsel(m3, bsel(m2, bsel(m1, b_ref[14], b_ref[12]),
                           bsel(m1, b_ref[10], b_ref[8])),
                  bsel(m2, bsel(m1, b_ref[6], b_ref[4]),
                       bsel(m1, b_ref[2], b_ref[0])))
    idx = (jnp.where(m3, 8, 0) | jnp.where(m2, 4, 0)
           | jnp.where(m1, 2, 0) | jnp.where(m0, 1, 0))
    yhat = bsel(m3, bsel(m2, bsel(m1, bsel(m0, c_ref[15], c_ref[14]),
                                  bsel(m0, c_ref[13], c_ref[12])),
                         bsel(m1, bsel(m0, c_ref[11], c_ref[10]),
                              bsel(m0, c_ref[9], c_ref[8]))),
                bsel(m2, bsel(m1, bsel(m0, c_ref[7], c_ref[6]),
                              bsel(m0, c_ref[5], c_ref[4])),
                     bsel(m1, bsel(m0, c_ref[3], c_ref[2]),
                          bsel(m0, c_ref[1], c_ref[0]))))
    idx_ref[...] = idx
    yhat_ref[...] = yhat.astype(jnp.bfloat16)


def _unrotate_kernel(yhat_ref, pi_ref, out_ref):
    out_ref[...] = jax.lax.dot_general(
        yhat_ref[...], pi_ref[...],
        dimension_numbers=(((1,), (1,)), ((), ())),
        preferred_element_type=jnp.float32,
    )


def kernel(x, Pi, centroids, boundaries):
    M, d = x.shape

    pi_bf = Pi.astype(jnp.bfloat16)

    idx, yhat = pl.pallas_call(
        _quant_matmul_kernel,
        grid=(M // BM1, d // BN1),
        in_specs=[
            pl.BlockSpec(memory_space=pltpu.SMEM),
            pl.BlockSpec(memory_space=pltpu.SMEM),
            pl.BlockSpec((BM1, d), lambda i, j: (i, 0)),
            pl.BlockSpec((d, BN1), lambda i, j: (0, j)),
        ],
        out_specs=[
            pl.BlockSpec((BM1, BN1), lambda i, j: (i, j)),
            pl.BlockSpec((BM1, BN1), lambda i, j: (i, j)),
        ],
        out_shape=[
            jax.ShapeDtypeStruct((M, d), jnp.int32),
            jax.ShapeDtypeStruct((M, d), jnp.bfloat16),
        ],
        compiler_params=pltpu.CompilerParams(
            dimension_semantics=("parallel", "arbitrary"),
        ),
    )(boundaries, centroids, x, pi_bf)

    x_hat = pl.pallas_call(
        _unrotate_kernel,
        grid=(M // BM2, d // BN2),
        in_specs=[
            pl.BlockSpec((BM2, d), lambda i, j: (i, 0)),
            pl.BlockSpec((BN2, d), lambda i, j: (j, 0)),
        ],
        out_specs=pl.BlockSpec((BM2, BN2), lambda i, j: (i, j)),
        out_shape=jax.ShapeDtypeStruct((M, d), jnp.float32),
        compiler_params=pltpu.CompilerParams(
            dimension_semantics=("parallel", "arbitrary"),
        ),
    )(yhat, pi_bf)

    return (x_hat, idx)
